# baseline (speedup 1.0000x reference)
import functools

import jax
import jax.numpy as jnp
from jax import lax
from jax.experimental import pallas as pl
from jax.experimental.pallas import tpu as pltpu
from jax.experimental.pallas import tpu_sc as plsc

NW = 32
C = 128


@functools.partial(jax.jit, static_argnums=(2, 3, 4, 5))
def _emb_lookup(idx1, table, S0, S1, D, B):
    mesh = plsc.VectorSubcoreMesh(core_axis_name="c", subcore_axis_name="s")
    rows_per_w = B // NW
    n_ch = rows_per_w // C

    @functools.partial(
        pl.kernel,
        mesh=mesh,
        out_type=jax.ShapeDtypeStruct((B, D), jnp.float32),
        scratch_types=[
            pltpu.VMEM((rows_per_w,), jnp.int32),
            pltpu.VMEM((2, C, D), jnp.float32),
            pltpu.SemaphoreType.DMA,
            pltpu.SemaphoreType.DMA,
            pltpu.SemaphoreType.DMA,
        ],
        compiler_params=pltpu.CompilerParams(use_tc_tiling_on_sc=False),
    )
    def emb(idx_hbm, table_hbm, out_hbm, idx_v, rows, ssem, gsem, wsem):
        wid = lax.axis_index("s") * 2 + lax.axis_index("c")
        base = wid * rows_per_w
        pltpu.async_copy(idx_hbm.at[pl.ds(base, rows_per_w)], idx_v,
                         ssem).wait()

        def gather(u):
            return pltpu.async_copy(
                table_hbm.at[idx_v.at[pl.ds(u * C, C)]], rows.at[u % 2], gsem)

        def write(u):
            return pltpu.async_copy(rows.at[u % 2],
                                    out_hbm.at[pl.ds(base + u * C, C)], wsem)

        hs = [None] * (n_ch + 1)
        ws = [None] * n_ch
        hs[0] = gather(0)
        for u in range(n_ch):
            if u >= 1:
                ws[u - 1].wait()
            if u + 1 < n_ch:
                hs[u + 1] = gather(u + 1)
            hs[u].wait()
            ws[u] = write(u)
        ws[n_ch - 1].wait()

    return emb(idx1, table)


def kernel(indices, table):
    S0, S1 = indices.shape
    V, D = table.shape
    B = S0 * S1
    assert B % (NW * C) == 0
    idx1 = indices.astype(jnp.int32).reshape(B)
    out2 = _emb_lookup(idx1, table, S0, S1, D, B)
    return out2.reshape(S0, S1, D)


# speedup vs baseline: 1.9106x; 1.9106x over previous
import functools

import jax
import jax.numpy as jnp
from jax import lax
from jax.experimental import pallas as pl
from jax.experimental.pallas import tpu as pltpu
from jax.experimental.pallas import tpu_sc as plsc

NW = 32
L = 16


@functools.partial(jax.jit, static_argnums=(2, 3, 4, 5))
def _emb_lookup(idx1, table3, S0, S1, D, B):
    mesh = plsc.VectorSubcoreMesh(core_axis_name="c", subcore_axis_name="s")
    rows_per_w = B // NW
    n_slabs = rows_per_w // S1
    n_grp = (S1 + L - 1) // L

    @functools.partial(
        pl.kernel,
        mesh=mesh,
        out_type=jax.ShapeDtypeStruct((S0, S1, D), jnp.float32),
        scratch_types=[
            pltpu.VMEM((rows_per_w + L,), jnp.int32),
            pltpu.VMEM((2, S1, D), jnp.float32),
            pltpu.SemaphoreType.DMA,
            pltpu.SemaphoreType.DMA((4,)),
            pltpu.SemaphoreType.DMA,
        ],
    )
    def emb(idx_hbm, table_hbm, out_hbm, idx_v, stage, ssem, gsems, wsem):
        wid = lax.axis_index("s") * 2 + lax.axis_index("c")
        base = wid * rows_per_w
        s0_base = wid * n_slabs
        pltpu.async_copy(idx_hbm.at[pl.ds(base, rows_per_w)],
                         idx_v.at[pl.ds(0, rows_per_w)], ssem).wait()

        def issue(j, b):
            for g in range(n_grp):
                vs = idx_v[pl.ds(j * S1 + g * L, L)]
                tv = vs >> 3
                rv = vs & 7
                for k in range(min(L, S1 - g * L)):
                    r = g * L + k
                    pltpu.async_copy(table_hbm.at[tv[k], rv[k]],
                                     stage.at[b, r],
                                     gsems.at[2 * b + r % 2])

        issue(0, 0)

        def pair(jh, carry):
            for b in range(2):
                j = jh * 2 + b

                @pl.when(j >= 1)
                def _():
                    pltpu.make_async_copy(stage.at[b], out_hbm.at[0],
                                          wsem).wait()

                @pl.when(j + 1 < n_slabs)
                def _():
                    issue(j + 1, (b + 1) % 2)

                for q in range(2 * b, 2 * b + 2):
                    for _ in range(S1 // 2):
                        pltpu.make_async_copy(table_hbm.at[0, 0],
                                              stage.at[0, 0],
                                              gsems.at[q]).wait()

                pltpu.async_copy(stage.at[b], out_hbm.at[s0_base + j], wsem)
            return carry

        lax.fori_loop(0, n_slabs // 2, pair, 0)
        pltpu.make_async_copy(stage.at[0], out_hbm.at[0], wsem).wait()

    return emb(idx1, table3)


def kernel(indices, table):
    S0, S1 = indices.shape
    V, D = table.shape
    B = S0 * S1
    assert B % (NW * S1) == 0 and V % 8 == 0 and D % L == 0
    assert (B // (NW * S1)) % 2 == 0
    idx1 = indices.astype(jnp.int32).reshape(B)
    table3 = table.reshape(V // 8, 8, D)
    return _emb_lookup(idx1, table3, S0, S1, D, B)
